# baseline (speedup 1.0000x reference)
import functools

import jax
import jax.numpy as jnp
from jax import lax
from jax.experimental import pallas as pl
from jax.experimental.pallas import tpu as pltpu
from jax.experimental.pallas import tpu_sc as plsc

_BATCH = 16384
_EMBED_DIM = 64
_CHUNK = 128
_LANES = 16
_DEPTH = 8


@functools.cache
def _build(num_user: int):
    info = plsc.get_sparse_core_info()
    num_workers = info.num_cores * info.num_subcores
    b_per_w = _BATCH // num_workers
    n_chunks = b_per_w // _CHUNK
    n_groups = b_per_w // _LANES
    groups_per_chunk = _CHUNK // _LANES
    mesh = plsc.VectorSubcoreMesh(core_axis_name="c", subcore_axis_name="s")

    @functools.partial(
        pl.kernel,
        mesh=mesh,
        out_type=jax.ShapeDtypeStruct((_EMBED_DIM, _BATCH), jnp.float32),
        scratch_types=[
            pltpu.VMEM((n_chunks, _CHUNK), jnp.int32),
            pltpu.VMEM((_DEPTH, _EMBED_DIM, _CHUNK), jnp.float32),
            pltpu.VMEM((2, _EMBED_DIM, _CHUNK), jnp.float32),
            [pltpu.SemaphoreType.DMA] * _DEPTH,
            pltpu.SemaphoreType.DMA,
        ],
        compiler_params=pltpu.CompilerParams(needs_layout_passes=False),
    )
    def gather_kernel(idx_hbm, table_hbm, out_hbm, idx_v, slots_v, stage_v,
                      sems, out_sem):
        wid = lax.axis_index("s") * info.num_cores + lax.axis_index("c")
        base = wid * b_per_w
        pltpu.sync_copy(idx_hbm.at[pl.ds(wid * n_chunks, n_chunks)], idx_v)

        lane16 = lax.iota(jnp.int32, _LANES)

        def fire(slot, r):
            colbase = pl.multiple_of((r >> 7) * _CHUNK, _CHUNK)
            pltpu.async_copy(
                table_hbm.at[:, pl.ds(colbase, _CHUNK)],
                slots_v.at[slot],
                sems[slot],
            )

        def wait(slot):
            pltpu.make_async_copy(
                table_hbm.at[:, pl.ds(0, _CHUNK)],
                slots_v.at[slot],
                sems[slot],
            ).wait()

        def load_group(g):
            return idx_v[g // groups_per_chunk,
                         pl.ds((g % groups_per_chunk) * _LANES, _LANES)]

        rvec0 = load_group(0)
        for l in range(_DEPTH):
            fire(l, rvec0[l])

        def writeback(par, chunk):
            colout = pl.multiple_of(base + chunk * _CHUNK, _CHUNK)
            return pltpu.async_copy(
                stage_v.at[par], out_hbm.at[:, pl.ds(colout, _CHUNK)],
                out_sem)

        def body(g, rvec):
            rvec_next = load_group(jnp.minimum(g + 1, n_groups - 1))
            chunk = g // groups_per_chunk
            par = chunk % 2

            @pl.when(jnp.logical_and(g % groups_per_chunk == 0,
                                     g >= 2 * groups_per_chunk))
            def _():
                pltpu.make_async_copy(
                    stage_v.at[par], out_hbm.at[:, pl.ds(0, _CHUNK)],
                    out_sem).wait()

            for l in range(_LANES):
                slot = l % _DEPTH
                wait(slot)
                r = rvec[l]
                col = jnp.broadcast_to(r & (_CHUNK - 1), (_LANES,))
                pos = (g % groups_per_chunk) * _LANES + l
                posv = jnp.broadcast_to(pos, (_LANES,))
                for k in range(_EMBED_DIM // _LANES):
                    rows = lane16 + k * _LANES
                    val = plsc.load_gather(slots_v.at[slot], [rows, col])
                    plsc.store_scatter(stage_v.at[par], [rows, posv], val)
                if l < _LANES - _DEPTH:
                    fire(slot, rvec[l + _DEPTH])
                else:

                    @pl.when(g < n_groups - 1)
                    def _():
                        fire(slot, rvec_next[l - (_LANES - _DEPTH)])

            @pl.when(g % groups_per_chunk == groups_per_chunk - 1)
            def _():
                writeback(par, chunk)

            return rvec_next

        lax.fori_loop(0, n_groups, body, rvec0)
        for par in range(2):
            pltpu.make_async_copy(
                stage_v.at[par], out_hbm.at[:, pl.ds(0, _CHUNK)],
                out_sem).wait()

    return gather_kernel


def kernel(user_fea, embedding_user):
    idx = user_fea[:, 0].astype(jnp.int32).reshape(_BATCH // _CHUNK, _CHUNK)
    out_t = _build(embedding_user.shape[0])(idx, embedding_user.T)
    return out_t.T


# speedup vs baseline: 1.0005x; 1.0005x over previous
import functools

import jax
import jax.numpy as jnp
from jax import lax
from jax.experimental import pallas as pl
from jax.experimental.pallas import tpu as pltpu
from jax.experimental.pallas import tpu_sc as plsc

_BATCH = 16384
_EMBED_DIM = 64
_CHUNK = 128
_LANES = 16
_DEPTH = 8


@functools.cache
def _build(num_user: int):
    info = plsc.get_sparse_core_info()
    num_workers = info.num_cores * info.num_subcores
    b_per_w = _BATCH // num_workers
    n_chunks = b_per_w // _CHUNK
    n_groups = b_per_w // _LANES
    groups_per_chunk = _CHUNK // _LANES
    mesh = plsc.VectorSubcoreMesh(core_axis_name="c", subcore_axis_name="s")

    @functools.partial(
        pl.kernel,
        mesh=mesh,
        out_type=jax.ShapeDtypeStruct((_EMBED_DIM, _BATCH), jnp.float32),
        scratch_types=[
            pltpu.VMEM((n_chunks, _CHUNK), jnp.int32),
            pltpu.VMEM((_DEPTH, _EMBED_DIM, _CHUNK), jnp.float32),
            pltpu.VMEM((_EMBED_DIM, _CHUNK), jnp.float32),
            [pltpu.SemaphoreType.DMA] * _DEPTH,
        ],
        compiler_params=pltpu.CompilerParams(needs_layout_passes=False),
    )
    def gather_kernel(idx_hbm, table_hbm, out_hbm, idx_v, slots_v, stage_v,
                      sems):
        wid = lax.axis_index("s") * info.num_cores + lax.axis_index("c")
        base = wid * b_per_w
        pltpu.sync_copy(idx_hbm.at[pl.ds(wid * n_chunks, n_chunks)], idx_v)

        lane16 = lax.iota(jnp.int32, _LANES)

        def fire(slot, r):
            colbase = pl.multiple_of((r >> 7) * _CHUNK, _CHUNK)
            pltpu.async_copy(
                table_hbm.at[:, pl.ds(colbase, _CHUNK)],
                slots_v.at[slot],
                sems[slot],
            )

        def wait(slot):
            pltpu.make_async_copy(
                table_hbm.at[:, pl.ds(0, _CHUNK)],
                slots_v.at[slot],
                sems[slot],
            ).wait()

        def load_group(g):
            return idx_v[g // groups_per_chunk,
                         pl.ds((g % groups_per_chunk) * _LANES, _LANES)]

        rvec0 = load_group(0)
        for l in range(_DEPTH):
            fire(l, rvec0[l])

        def body(g, rvec):
            rvec_next = load_group(jnp.minimum(g + 1, n_groups - 1))
            for l in range(_LANES):
                slot = l % _DEPTH
                wait(slot)
                r = rvec[l]
                col = jnp.broadcast_to(r & (_CHUNK - 1), (_LANES,))
                pos = (g % groups_per_chunk) * _LANES + l
                posv = jnp.broadcast_to(pos, (_LANES,))
                for k in range(_EMBED_DIM // _LANES):
                    rows = lane16 + k * _LANES
                    val = plsc.load_gather(slots_v.at[slot], [rows, col])
                    plsc.store_scatter(stage_v, [rows, posv], val)
                if l < _LANES - _DEPTH:
                    fire(slot, rvec[l + _DEPTH])
                else:

                    @pl.when(g < n_groups - 1)
                    def _():
                        fire(slot, rvec_next[l - (_LANES - _DEPTH)])

            @pl.when(g % groups_per_chunk == groups_per_chunk - 1)
            def _():
                colout = pl.multiple_of(
                    base + (g // groups_per_chunk) * _CHUNK, _CHUNK)
                pltpu.sync_copy(
                    stage_v, out_hbm.at[:, pl.ds(colout, _CHUNK)])

            return rvec_next

        lax.fori_loop(0, n_groups, body, rvec0)

    return gather_kernel


def kernel(user_fea, embedding_user):
    idx = user_fea[:, 0].astype(jnp.int32).reshape(_BATCH // _CHUNK, _CHUNK)
    out_t = _build(embedding_user.shape[0])(idx, embedding_user.T)
    return out_t.T


# speedup vs baseline: 1.0005x; 1.0001x over previous
import functools

import jax
import jax.numpy as jnp
from jax import lax
from jax.experimental import pallas as pl
from jax.experimental.pallas import tpu as pltpu
from jax.experimental.pallas import tpu_sc as plsc

_BATCH = 16384
_EMBED_DIM = 64
_CHUNK = 128
_LANES = 16
_DEPTH = 8


@functools.cache
def _build(num_user: int):
    info = plsc.get_sparse_core_info()
    num_workers = info.num_cores * info.num_subcores
    b_per_w = _BATCH // num_workers
    n_chunks = b_per_w // _CHUNK
    n_groups = b_per_w // _LANES
    groups_per_chunk = _CHUNK // _LANES
    mesh = plsc.VectorSubcoreMesh(core_axis_name="c", subcore_axis_name="s")

    @functools.partial(
        pl.kernel,
        mesh=mesh,
        out_type=jax.ShapeDtypeStruct((_EMBED_DIM, _BATCH), jnp.float32),
        scratch_types=[
            pltpu.VMEM((n_chunks, _CHUNK), jnp.int32),
            pltpu.VMEM((_DEPTH, _EMBED_DIM, _CHUNK), jnp.float32),
            pltpu.VMEM((_EMBED_DIM, _CHUNK), jnp.float32),
            [pltpu.SemaphoreType.DMA] * _DEPTH,
        ],
        compiler_params=pltpu.CompilerParams(needs_layout_passes=False),
    )
    def gather_kernel(idx_hbm, table_hbm, out_hbm, idx_v, slots_v, stage_v,
                      sems):
        wid = lax.axis_index("s") * info.num_cores + lax.axis_index("c")
        base = wid * b_per_w
        pltpu.sync_copy(idx_hbm.at[pl.ds(wid * n_chunks, n_chunks)], idx_v)

        lane16 = lax.iota(jnp.int32, _LANES)

        def fire(slot, r):
            colbase = pl.multiple_of((r >> 7) * _CHUNK, _CHUNK)
            half = _EMBED_DIM // 2
            for h in range(2):
                pltpu.async_copy(
                    table_hbm.at[pl.ds(h * half, half),
                                 pl.ds(colbase, _CHUNK)],
                    slots_v.at[slot].at[pl.ds(h * half, half)],
                    sems[slot],
                )

        def wait(slot):
            pltpu.make_async_copy(
                table_hbm.at[:, pl.ds(0, _CHUNK)],
                slots_v.at[slot],
                sems[slot],
            ).wait()

        def load_group(g):
            return idx_v[g // groups_per_chunk,
                         pl.ds((g % groups_per_chunk) * _LANES, _LANES)]

        rvec0 = load_group(0)
        for l in range(_DEPTH):
            fire(l, rvec0[l])

        def body(g, rvec):
            rvec_next = load_group(jnp.minimum(g + 1, n_groups - 1))
            for l in range(_LANES):
                slot = l % _DEPTH
                wait(slot)
                r = rvec[l]
                col = jnp.broadcast_to(r & (_CHUNK - 1), (_LANES,))
                pos = (g % groups_per_chunk) * _LANES + l
                posv = jnp.broadcast_to(pos, (_LANES,))
                for k in range(_EMBED_DIM // _LANES):
                    rows = lane16 + k * _LANES
                    val = plsc.load_gather(slots_v.at[slot], [rows, col])
                    plsc.store_scatter(stage_v, [rows, posv], val)
                if l < _LANES - _DEPTH:
                    fire(slot, rvec[l + _DEPTH])
                else:

                    @pl.when(g < n_groups - 1)
                    def _():
                        fire(slot, rvec_next[l - (_LANES - _DEPTH)])

            @pl.when(g % groups_per_chunk == groups_per_chunk - 1)
            def _():
                colout = pl.multiple_of(
                    base + (g // groups_per_chunk) * _CHUNK, _CHUNK)
                pltpu.sync_copy(
                    stage_v, out_hbm.at[:, pl.ds(colout, _CHUNK)])

            return rvec_next

        lax.fori_loop(0, n_groups, body, rvec0)

    return gather_kernel


def kernel(user_fea, embedding_user):
    idx = user_fea[:, 0].astype(jnp.int32).reshape(_BATCH // _CHUNK, _CHUNK)
    out_t = _build(embedding_user.shape[0])(idx, embedding_user.T)
    return out_t.T


# speedup vs baseline: 1.0046x; 1.0040x over previous
import functools

import jax
import jax.numpy as jnp
from jax import lax
from jax.experimental import pallas as pl
from jax.experimental.pallas import tpu as pltpu
from jax.experimental.pallas import tpu_sc as plsc

_BATCH = 16384
_EMBED_DIM = 64
_CHUNK = 128
_LANES = 16
_DEPTH = 8


@functools.cache
def _build(num_user: int):
    info = plsc.get_sparse_core_info()
    num_workers = info.num_cores * info.num_subcores
    b_per_w = _BATCH // num_workers
    n_chunks = b_per_w // _CHUNK
    n_groups = b_per_w // _LANES
    groups_per_chunk = _CHUNK // _LANES
    mesh = plsc.VectorSubcoreMesh(core_axis_name="c", subcore_axis_name="s")

    @functools.partial(
        pl.kernel,
        mesh=mesh,
        out_type=jax.ShapeDtypeStruct((_EMBED_DIM, _BATCH), jnp.float32),
        scratch_types=[
            pltpu.VMEM((n_chunks, _CHUNK), jnp.int32),
            pltpu.VMEM((_DEPTH, _EMBED_DIM, _CHUNK), jnp.float32),
            pltpu.VMEM((_EMBED_DIM, _CHUNK), jnp.float32),
            [pltpu.SemaphoreType.DMA] * _DEPTH,
        ],
        compiler_params=pltpu.CompilerParams(needs_layout_passes=False),
    )
    def gather_kernel(idx_hbm, table_hbm, out_hbm, idx_v, slots_v, stage_v,
                      sems):
        wid = lax.axis_index("s") * info.num_cores + lax.axis_index("c")
        base = wid * b_per_w
        pltpu.sync_copy(idx_hbm.at[pl.ds(wid * n_chunks, n_chunks)], idx_v)

        lane16 = lax.iota(jnp.int32, _LANES)

        def fire(slot, r):
            colbase = pl.multiple_of((r >> 7) * _CHUNK, _CHUNK)
            pltpu.async_copy(
                table_hbm.at[:, pl.ds(colbase, _CHUNK)],
                slots_v.at[slot],
                sems[slot],
            )

        def wait(slot):
            pltpu.make_async_copy(
                table_hbm.at[:, pl.ds(0, _CHUNK)],
                slots_v.at[slot],
                sems[slot],
            ).wait()

        def load_group(g):
            return idx_v[g // groups_per_chunk,
                         pl.ds((g % groups_per_chunk) * _LANES, _LANES)]

        rvec0 = load_group(0)
        for l in range(_DEPTH):
            fire(l, rvec0[l])

        def body(g, rvec):
            rvec_next = load_group(jnp.minimum(g + 1, n_groups - 1))
            for l in range(_LANES):
                slot = l % _DEPTH
                wait(slot)
                r = rvec[l]
                col = jnp.broadcast_to(r & (_CHUNK - 1), (_LANES,))
                pos = (g % groups_per_chunk) * _LANES + l
                posv = jnp.broadcast_to(pos, (_LANES,))
                for k in range(_EMBED_DIM // _LANES):
                    rows = lane16 + k * _LANES
                    val = plsc.load_gather(slots_v.at[slot], [rows, col])
                    plsc.store_scatter(stage_v, [rows, posv], val)
                if l < _LANES - _DEPTH:
                    fire(slot, rvec[l + _DEPTH])
                else:

                    @pl.when(g < n_groups - 1)
                    def _():
                        fire(slot, rvec_next[l - (_LANES - _DEPTH)])

            @pl.when(g % groups_per_chunk == groups_per_chunk - 1)
            def _():
                colout = pl.multiple_of(
                    base + (g // groups_per_chunk) * _CHUNK, _CHUNK)
                pltpu.sync_copy(
                    stage_v, out_hbm.at[:, pl.ds(colout, _CHUNK)])

            return rvec_next

        lax.fori_loop(0, n_groups, body, rvec0)

    return gather_kernel


def kernel(user_fea, embedding_user):
    idx = user_fea[:, 0].astype(jnp.int32).reshape(_BATCH // _CHUNK, _CHUNK)
    out_t = _build(embedding_user.shape[0])(idx, embedding_user.T)
    return out_t.T
